# baseline (speedup 1.0000x reference)
import functools

import jax
import jax.numpy as jnp
from jax.experimental import pallas as pl
from jax.experimental.pallas import tpu as pltpu

D_MODEL = 2048
N_EXP = 16
BT = 256
BIG = 2048
NBIG = 7
TOK_D, TOK_E, TOK_F, TOK_G = 14336, 15360, 15872, 16128


def _top2_softmax_tail(logits, w_ref, i_ref, acc_ref, aux_ref, step, nsteps, inv_t):
    ids = jax.lax.broadcasted_iota(jnp.int32, logits.shape, 1)
    m1 = jnp.max(logits, axis=1, keepdims=True)
    i1 = jnp.min(jnp.where(logits == m1, ids, N_EXP), axis=1, keepdims=True)
    masked = jnp.where(ids == i1, -jnp.inf, logits)
    m2 = jnp.max(masked, axis=1, keepdims=True)
    i2 = jnp.min(jnp.where(masked == m2, ids, N_EXP), axis=1, keepdims=True)

    t = jnp.exp(m2 - m1)
    w1 = 1.0 / (1.0 + t)
    w2 = t / (1.0 + t)
    w_ref[...] = jnp.concatenate([w1, w2], axis=1)
    i_ref[...] = jnp.concatenate([i1, i2], axis=1).astype(jnp.int32)

    p = jnp.exp(logits - m1)
    probs = p / jnp.sum(p, axis=1, keepdims=True)
    part = jnp.sum(probs, axis=0, keepdims=True)

    @pl.when(step == 0)
    def _():
        acc_ref[...] = jnp.zeros_like(acc_ref)

    acc_ref[...] += part

    @pl.when(step == nsteps - 1)
    def _():
        mean = acc_ref[...] * inv_t
        aux_ref[...] = jnp.sum(mean * mean, keepdims=True) * float(N_EXP)


def _router_block(x_hbm, wt_ref, w_ref, i_ref, acc_ref, aux_ref,
                  big_ref, d_ref, e_ref, f_ref, g_ref, logit_ref,
                  sem_big, sem_sm, *, nsteps, inv_t):
    step = pl.program_id(0)

    def copy_big(chunk, slot):
        pltpu.make_async_copy(
            x_hbm.at[pl.ds(chunk * BIG, BIG), :],
            big_ref.at[slot],
            sem_big.at[slot],
        ).start()

    def copy_small(tok, n, dst, sem):
        pltpu.make_async_copy(
            x_hbm.at[pl.ds(tok, n), :], dst, sem,
        ).start()

    @pl.when(step == 0)
    def _():
        copy_big(0, 0)
        copy_big(1, 1)

    @pl.when((step >= 8) & (step <= 40) & (step % 8 == 0))
    def _():
        chunk = step // 8 + 1
        copy_big(chunk, jax.lax.rem(chunk, 2))

    @pl.when(step == 40)
    def _():
        copy_small(TOK_D, 1024, d_ref, sem_sm.at[0])
        copy_small(TOK_E, 512, e_ref, sem_sm.at[1])
        copy_small(TOK_F, 256, f_ref, sem_sm.at[2])
        copy_small(TOK_G, 256, g_ref, sem_sm.at[3])

    @pl.when((step < 56) & (step % 8 == 0))
    def _():
        slot = jax.lax.rem(step // 8, 2)
        pltpu.make_async_copy(
            x_hbm.at[pl.ds((step // 8) * BIG, BIG), :],
            big_ref.at[slot],
            sem_big.at[slot],
        ).wait()

    @pl.when(step == 56)
    def _():
        pltpu.make_async_copy(
            x_hbm.at[pl.ds(TOK_D, 1024), :], d_ref, sem_sm.at[0]).wait()

    @pl.when(step == 60)
    def _():
        pltpu.make_async_copy(
            x_hbm.at[pl.ds(TOK_E, 512), :], e_ref, sem_sm.at[1]).wait()

    @pl.when(step == 62)
    def _():
        pltpu.make_async_copy(
            x_hbm.at[pl.ds(TOK_F, 256), :], f_ref, sem_sm.at[2]).wait()

    @pl.when(step == 63)
    def _():
        pltpu.make_async_copy(
            x_hbm.at[pl.ds(TOK_G, 256), :], g_ref, sem_sm.at[3]).wait()

    wt = wt_ref[...]

    @pl.when(step < 56)
    def _():
        slot = jax.lax.rem(step // 8, 2)
        off = jax.lax.rem(step, 8) * BT
        logit_ref[...] = jnp.dot(big_ref[slot, pl.ds(off, BT), :], wt,
                                 preferred_element_type=jnp.float32)

    @pl.when((step >= 56) & (step < 60))
    def _():
        off = (step - 56) * BT
        logit_ref[...] = jnp.dot(d_ref[pl.ds(off, BT), :], wt,
                                 preferred_element_type=jnp.float32)

    @pl.when((step >= 60) & (step < 62))
    def _():
        off = (step - 60) * BT
        logit_ref[...] = jnp.dot(e_ref[pl.ds(off, BT), :], wt,
                                 preferred_element_type=jnp.float32)

    @pl.when(step == 62)
    def _():
        logit_ref[...] = jnp.dot(f_ref[...], wt,
                                 preferred_element_type=jnp.float32)

    @pl.when(step == 63)
    def _():
        logit_ref[...] = jnp.dot(g_ref[...], wt,
                                 preferred_element_type=jnp.float32)

    _top2_softmax_tail(logit_ref[...], w_ref, i_ref, acc_ref, aux_ref,
                       step, nsteps, inv_t)


def kernel(x, W):
    b, tt, d = x.shape
    total = b * tt
    xf = x.reshape(total, d)
    wt = W.T
    nsteps = total // BT

    body = functools.partial(_router_block, nsteps=nsteps, inv_t=1.0 / total)
    weights, indices, _, aux = pl.pallas_call(
        body,
        grid=(nsteps,),
        in_specs=[
            pl.BlockSpec(memory_space=pl.ANY),
            pl.BlockSpec((d, N_EXP), lambda i: (0, 0)),
        ],
        out_specs=[
            pl.BlockSpec((BT, 2), lambda i: (i, 0)),
            pl.BlockSpec((BT, 2), lambda i: (i, 0)),
            pl.BlockSpec((1, N_EXP), lambda i: (0, 0)),
            pl.BlockSpec((1, 1), lambda i: (0, 0)),
        ],
        out_shape=[
            jax.ShapeDtypeStruct((total, 2), jnp.float32),
            jax.ShapeDtypeStruct((total, 2), jnp.int32),
            jax.ShapeDtypeStruct((1, N_EXP), jnp.float32),
            jax.ShapeDtypeStruct((1, 1), jnp.float32),
        ],
        scratch_shapes=[
            pltpu.VMEM((2, BIG, D_MODEL), jnp.float32),
            pltpu.VMEM((1024, D_MODEL), jnp.float32),
            pltpu.VMEM((512, D_MODEL), jnp.float32),
            pltpu.VMEM((256, D_MODEL), jnp.float32),
            pltpu.VMEM((256, D_MODEL), jnp.float32),
            pltpu.VMEM((BT, N_EXP), jnp.float32),
            pltpu.SemaphoreType.DMA((2,)),
            pltpu.SemaphoreType.DMA((4,)),
        ],
    )(xf, wt)

    return (weights.reshape(b, tt, 2), indices.reshape(b, tt, 2), aux[0, 0])


# speedup vs baseline: 1.2131x; 1.2131x over previous
import functools

import jax
import jax.numpy as jnp
from jax.experimental import pallas as pl
from jax.experimental.pallas import tpu as pltpu

D_MODEL = 2048
N_EXP = 16
BT = 256
NBUF = 8


def _router_block(x_hbm, wt_ref, w_ref, i_ref, acc_ref, aux_ref,
                  buf_ref, sem, *, nsteps, inv_t):
    step = pl.program_id(0)

    def start_copy(src_step, slot):
        pltpu.make_async_copy(
            x_hbm.at[pl.ds(src_step * BT, BT), :],
            buf_ref.at[slot],
            sem.at[slot],
        ).start()

    @pl.when(step == 0)
    def _():
        for j in range(NBUF):
            start_copy(j, j)

    slot = jax.lax.rem(step, NBUF)
    pltpu.make_async_copy(
        x_hbm.at[pl.ds(step * BT, BT), :],
        buf_ref.at[slot],
        sem.at[slot],
    ).wait()

    xb = buf_ref[slot]

    @pl.when(step + NBUF < nsteps)
    def _():
        start_copy(step + NBUF, slot)

    logits = jnp.dot(xb, wt_ref[...], preferred_element_type=jnp.float32)

    ids = jax.lax.broadcasted_iota(jnp.int32, logits.shape, 1)
    m1 = jnp.max(logits, axis=1, keepdims=True)
    i1 = jnp.min(jnp.where(logits == m1, ids, N_EXP), axis=1, keepdims=True)
    masked = jnp.where(ids == i1, -jnp.inf, logits)
    m2 = jnp.max(masked, axis=1, keepdims=True)
    i2 = jnp.min(jnp.where(masked == m2, ids, N_EXP), axis=1, keepdims=True)

    t = jnp.exp(m2 - m1)
    w1 = 1.0 / (1.0 + t)
    w2 = t / (1.0 + t)
    w_ref[...] = jnp.concatenate([w1, w2], axis=1)
    i_ref[...] = jnp.concatenate([i1, i2], axis=1).astype(jnp.int32)

    p = jnp.exp(logits - m1)
    probs = p / jnp.sum(p, axis=1, keepdims=True)
    part = jnp.sum(probs, axis=0, keepdims=True)

    @pl.when(step == 0)
    def _():
        acc_ref[...] = jnp.zeros_like(acc_ref)

    acc_ref[...] += part

    @pl.when(step == nsteps - 1)
    def _():
        mean = acc_ref[...] * inv_t
        aux_ref[...] = jnp.sum(mean * mean, keepdims=True) * float(N_EXP)


def kernel(x, W):
    b, tt, d = x.shape
    total = b * tt
    xf = x.reshape(total, d)
    wt = W.T
    nsteps = total // BT

    body = functools.partial(_router_block, nsteps=nsteps, inv_t=1.0 / total)
    weights, indices, _, aux = pl.pallas_call(
        body,
        grid=(nsteps,),
        in_specs=[
            pl.BlockSpec(memory_space=pl.ANY),
            pl.BlockSpec((d, N_EXP), lambda i: (0, 0)),
        ],
        out_specs=[
            pl.BlockSpec((BT, 2), lambda i: (i, 0)),
            pl.BlockSpec((BT, 2), lambda i: (i, 0)),
            pl.BlockSpec((1, N_EXP), lambda i: (0, 0)),
            pl.BlockSpec((1, 1), lambda i: (0, 0)),
        ],
        out_shape=[
            jax.ShapeDtypeStruct((total, 2), jnp.float32),
            jax.ShapeDtypeStruct((total, 2), jnp.int32),
            jax.ShapeDtypeStruct((1, N_EXP), jnp.float32),
            jax.ShapeDtypeStruct((1, 1), jnp.float32),
        ],
        scratch_shapes=[
            pltpu.VMEM((NBUF, BT, D_MODEL), jnp.float32),
            pltpu.SemaphoreType.DMA((NBUF,)),
        ],
    )(xf, wt)

    return (weights.reshape(b, tt, 2), indices.reshape(b, tt, 2), aux[0, 0])


# speedup vs baseline: 1.4453x; 1.1915x over previous
import functools

import jax
import jax.numpy as jnp
from jax.experimental import pallas as pl
from jax.experimental.pallas import tpu as pltpu

D_MODEL = 2048
N_EXP = 16
BT = 512
NBUF = 6


def _router_block(x_hbm, wt_ref, w_ref, i_ref, aux_ref, buf_ref, acc_ref, sem,
                  *, nsteps, inv_t):
    step = pl.program_id(0)

    def start_copy(src_step, slot):
        pltpu.make_async_copy(
            x_hbm.at[pl.ds(src_step * BT, BT), :],
            buf_ref.at[slot],
            sem.at[slot],
        ).start()

    @pl.when(step == 0)
    def _():
        for j in range(NBUF):
            start_copy(j, j)

    slot = jax.lax.rem(step, NBUF)
    pltpu.make_async_copy(
        x_hbm.at[pl.ds(step * BT, BT), :],
        buf_ref.at[slot],
        sem.at[slot],
    ).wait()

    xb = buf_ref[slot]

    @pl.when(step + NBUF < nsteps)
    def _():
        start_copy(step + NBUF, slot)

    logits = jnp.dot(xb, wt_ref[...], preferred_element_type=jnp.float32)

    ids = jax.lax.broadcasted_iota(jnp.int32, logits.shape, 1)
    m1 = jnp.max(logits, axis=1, keepdims=True)
    i1 = jnp.min(jnp.where(logits == m1, ids, N_EXP), axis=1, keepdims=True)
    masked = jnp.where(ids == i1, -jnp.inf, logits)
    m2 = jnp.max(masked, axis=1, keepdims=True)
    i2 = jnp.min(jnp.where(masked == m2, ids, N_EXP), axis=1, keepdims=True)

    t = jnp.exp(m2 - m1)
    w1 = 1.0 / (1.0 + t)
    w2 = t / (1.0 + t)
    base = step * BT
    w_ref[pl.ds(base, BT), :] = jnp.concatenate([w1, w2], axis=1)
    i_ref[pl.ds(base, BT), :] = jnp.concatenate([i1, i2], axis=1).astype(jnp.int32)

    p = jnp.exp(logits - m1)
    probs = p / jnp.sum(p, axis=1, keepdims=True)
    part = jnp.sum(probs, axis=0, keepdims=True)

    @pl.when(step == 0)
    def _():
        acc_ref[...] = jnp.zeros_like(acc_ref)

    acc_ref[...] += part

    @pl.when(step == nsteps - 1)
    def _():
        mean = acc_ref[...] * inv_t
        aux_ref[...] = jnp.sum(mean * mean, keepdims=True) * float(N_EXP)


def kernel(x, W):
    b, tt, d = x.shape
    total = b * tt
    xf = x.reshape(total, d)
    wt = W.T
    nsteps = total // BT

    body = functools.partial(_router_block, nsteps=nsteps, inv_t=1.0 / total)
    weights, indices, aux = pl.pallas_call(
        body,
        grid=(nsteps,),
        in_specs=[
            pl.BlockSpec(memory_space=pl.ANY),
            pl.BlockSpec((d, N_EXP), lambda i: (0, 0)),
        ],
        out_specs=[
            pl.BlockSpec(memory_space=pltpu.VMEM),
            pl.BlockSpec(memory_space=pltpu.VMEM),
            pl.BlockSpec(memory_space=pltpu.VMEM),
        ],
        out_shape=[
            jax.ShapeDtypeStruct((total, 2), jnp.float32),
            jax.ShapeDtypeStruct((total, 2), jnp.int32),
            jax.ShapeDtypeStruct((1, 1), jnp.float32),
        ],
        scratch_shapes=[
            pltpu.VMEM((NBUF, BT, D_MODEL), jnp.float32),
            pltpu.VMEM((1, N_EXP), jnp.float32),
            pltpu.SemaphoreType.DMA((NBUF,)),
        ],
    )(xf, wt)

    return (weights.reshape(b, tt, 2), indices.reshape(b, tt, 2), aux[0, 0])


# speedup vs baseline: 1.5556x; 1.0763x over previous
import functools

import jax
import jax.numpy as jnp
from jax.experimental import pallas as pl
from jax.experimental.pallas import tpu as pltpu

D_MODEL = 2048
N_EXP = 16
BT = 512
NBUF = 4


def _router_block(x_hbm, wt_ref, w_ref, i_ref, acc_ref, aux_ref,
                  buf_ref, sem, *, nsteps, inv_t):
    step = pl.program_id(0)

    def start_copy(src_step, slot):
        pltpu.make_async_copy(
            x_hbm.at[pl.ds(src_step * BT, BT), :],
            buf_ref.at[slot],
            sem.at[slot],
        ).start()

    @pl.when(step == 0)
    def _():
        for j in range(NBUF):
            start_copy(j, j)

    slot = jax.lax.rem(step, NBUF)
    pltpu.make_async_copy(
        x_hbm.at[pl.ds(step * BT, BT), :],
        buf_ref.at[slot],
        sem.at[slot],
    ).wait()

    xb = buf_ref[slot]

    @pl.when(step + NBUF < nsteps)
    def _():
        start_copy(step + NBUF, slot)

    logits = jnp.dot(xb, wt_ref[...], preferred_element_type=jnp.float32)

    ids = jax.lax.broadcasted_iota(jnp.int32, logits.shape, 1)
    m1 = jnp.max(logits, axis=1, keepdims=True)
    i1 = jnp.min(jnp.where(logits == m1, ids, N_EXP), axis=1, keepdims=True)
    masked = jnp.where(ids == i1, -jnp.inf, logits)
    m2 = jnp.max(masked, axis=1, keepdims=True)
    i2 = jnp.min(jnp.where(masked == m2, ids, N_EXP), axis=1, keepdims=True)

    t = jnp.exp(m2 - m1)
    w1 = 1.0 / (1.0 + t)
    w2 = t / (1.0 + t)
    w_ref[...] = jnp.concatenate([w1, w2], axis=1)
    i_ref[...] = jnp.concatenate([i1, i2], axis=1).astype(jnp.int32)

    p = jnp.exp(logits - m1)
    probs = p / jnp.sum(p, axis=1, keepdims=True)
    part = jnp.sum(probs, axis=0, keepdims=True)

    @pl.when(step == 0)
    def _():
        acc_ref[...] = jnp.zeros_like(acc_ref)

    acc_ref[...] += part

    @pl.when(step == nsteps - 1)
    def _():
        mean = acc_ref[...] * inv_t
        aux_ref[...] = jnp.sum(mean * mean, keepdims=True) * float(N_EXP)


def kernel(x, W):
    b, tt, d = x.shape
    total = b * tt
    xf = x.reshape(total, d)
    wt = W.T
    nsteps = total // BT

    body = functools.partial(_router_block, nsteps=nsteps, inv_t=1.0 / total)
    weights, indices, _, aux = pl.pallas_call(
        body,
        grid=(nsteps,),
        in_specs=[
            pl.BlockSpec(memory_space=pl.ANY),
            pl.BlockSpec((d, N_EXP), lambda i: (0, 0)),
        ],
        out_specs=[
            pl.BlockSpec((BT, 2), lambda i: (i, 0)),
            pl.BlockSpec((BT, 2), lambda i: (i, 0)),
            pl.BlockSpec((1, N_EXP), lambda i: (0, 0)),
            pl.BlockSpec((1, 1), lambda i: (0, 0)),
        ],
        out_shape=[
            jax.ShapeDtypeStruct((total, 2), jnp.float32),
            jax.ShapeDtypeStruct((total, 2), jnp.int32),
            jax.ShapeDtypeStruct((1, N_EXP), jnp.float32),
            jax.ShapeDtypeStruct((1, 1), jnp.float32),
        ],
        scratch_shapes=[
            pltpu.VMEM((NBUF, BT, D_MODEL), jnp.float32),
            pltpu.SemaphoreType.DMA((NBUF,)),
        ],
    )(xf, wt)

    return (weights.reshape(b, tt, 2), indices.reshape(b, tt, 2), aux[0, 0])
